# baseline (speedup 1.0000x reference)
import jax
import jax.numpy as jnp
from jax.experimental import pallas as pl
from jax.experimental.pallas import tpu as pltpu

_N_TOKENS = 32768
_HIDDEN = 4096
_NUM_EXPERTS = 64
_TOP_K = 8
_BR = 512


def _gating_kernel(x_ref, w_ref, logits_ref, wts_ref, idx_ref):
    x = x_ref[...]
    w = w_ref[...]
    logits = jax.lax.dot_general(
        x, w, (((1,), (1,)), ((), ())), preferred_element_type=jnp.float32
    )
    logits_ref[...] = logits

    lane = jax.lax.broadcasted_iota(jnp.int32, (_BR, _NUM_EXPERTS), 1)
    bits = jax.lax.bitcast_convert_type(logits, jnp.int32)
    mono = bits ^ ((bits >> 31) & jnp.int32(0x7FFFFFFF))
    key = (mono & jnp.int32(~63)) | (jnp.int32(63) - lane)

    vals, idxs = [], []
    for _ in range(_TOP_K):
        kmax = jnp.max(key, axis=1, keepdims=True)
        idxs.append(jnp.int32(63) - (kmax & jnp.int32(63)))
        kv = kmax & jnp.int32(~63)
        vbits = kv ^ ((kv >> 31) & jnp.int32(0x7FFFFFFF))
        vals.append(jax.lax.bitcast_convert_type(vbits, jnp.float32))
        key = jnp.where(key == kmax, jnp.int32(-(2**31)), key)

    topv = jnp.concatenate(vals, axis=1)
    topi = jnp.concatenate(idxs, axis=1)
    e = jnp.exp(topv - topv[:, :1])
    wts_ref[...] = e / jnp.sum(e, axis=1, keepdims=True)
    idx_ref[...] = topi


def kernel(x, W):
    grid = (_N_TOKENS // _BR,)
    logits, wts, idx = pl.pallas_call(
        _gating_kernel,
        grid=grid,
        in_specs=[
            pl.BlockSpec((_BR, _HIDDEN), lambda i: (i, 0)),
            pl.BlockSpec((_NUM_EXPERTS, _HIDDEN), lambda i: (0, 0)),
        ],
        out_specs=[
            pl.BlockSpec((_BR, _NUM_EXPERTS), lambda i: (i, 0)),
            pl.BlockSpec((_BR, _TOP_K), lambda i: (i, 0)),
            pl.BlockSpec((_BR, _TOP_K), lambda i: (i, 0)),
        ],
        out_shape=[
            jax.ShapeDtypeStruct((_N_TOKENS, _NUM_EXPERTS), jnp.float32),
            jax.ShapeDtypeStruct((_N_TOKENS, _TOP_K), jnp.float32),
            jax.ShapeDtypeStruct((_N_TOKENS, _TOP_K), jnp.int32),
        ],
        compiler_params=pltpu.CompilerParams(
            dimension_semantics=("parallel",),
        ),
    )(x, W)
    return (wts, idx, logits)


# speedup vs baseline: 1.0948x; 1.0948x over previous
import jax
import jax.numpy as jnp
from jax.experimental import pallas as pl
from jax.experimental.pallas import tpu as pltpu

_N_TOKENS = 32768
_HIDDEN = 4096
_NUM_EXPERTS = 64
_TOP_K = 8
_BR = 1024


def _gating_kernel(x_ref, w_ref, logits_ref, wts_ref, idx_ref):
    x = x_ref[...]
    w = w_ref[...]
    logits = jax.lax.dot_general(
        x, w, (((1,), (1,)), ((), ())), preferred_element_type=jnp.float32
    )
    logits_ref[...] = logits

    lane = jax.lax.broadcasted_iota(jnp.int32, (_BR, _NUM_EXPERTS), 1)
    bits = jax.lax.bitcast_convert_type(logits, jnp.int32)
    mono = bits ^ ((bits >> 31) & jnp.int32(0x7FFFFFFF))
    key = (mono & jnp.int32(~63)) | (jnp.int32(63) - lane)

    vals, idxs = [], []
    for _ in range(_TOP_K):
        kmax = jnp.max(key, axis=1, keepdims=True)
        idxs.append(jnp.int32(63) - (kmax & jnp.int32(63)))
        kv = kmax & jnp.int32(~63)
        vbits = kv ^ ((kv >> 31) & jnp.int32(0x7FFFFFFF))
        vals.append(jax.lax.bitcast_convert_type(vbits, jnp.float32))
        key = jnp.where(key == kmax, jnp.int32(-(2**31)), key)

    topv = jnp.concatenate(vals, axis=1)
    topi = jnp.concatenate(idxs, axis=1)
    e = jnp.exp(topv - topv[:, :1])
    wts_ref[...] = e / jnp.sum(e, axis=1, keepdims=True)
    idx_ref[...] = topi


def kernel(x, W):
    grid = (_N_TOKENS // _BR,)
    logits, wts, idx = pl.pallas_call(
        _gating_kernel,
        grid=grid,
        in_specs=[
            pl.BlockSpec((_BR, _HIDDEN), lambda i: (i, 0)),
            pl.BlockSpec((_NUM_EXPERTS, _HIDDEN), lambda i: (0, 0)),
        ],
        out_specs=[
            pl.BlockSpec((_BR, _NUM_EXPERTS), lambda i: (i, 0)),
            pl.BlockSpec((_BR, _TOP_K), lambda i: (i, 0)),
            pl.BlockSpec((_BR, _TOP_K), lambda i: (i, 0)),
        ],
        out_shape=[
            jax.ShapeDtypeStruct((_N_TOKENS, _NUM_EXPERTS), jnp.float32),
            jax.ShapeDtypeStruct((_N_TOKENS, _TOP_K), jnp.float32),
            jax.ShapeDtypeStruct((_N_TOKENS, _TOP_K), jnp.int32),
        ],
        compiler_params=pltpu.CompilerParams(
            dimension_semantics=("parallel",),
        ),
    )(x, W)
    return (wts, idx, logits)


# speedup vs baseline: 1.3161x; 1.2022x over previous
import jax
import jax.numpy as jnp
from jax.experimental import pallas as pl
from jax.experimental.pallas import tpu as pltpu

_N_TOKENS = 32768
_HIDDEN = 4096
_NUM_EXPERTS = 64
_TOP_K = 8
_BR = 1024


def _gating_kernel(x_ref, w_ref, logits_ref, wts_ref, idx_ref):
    x = x_ref[...]
    w = w_ref[...]
    lt = jax.lax.dot_general(
        w, x, (((1,), (1,)), ((), ())), preferred_element_type=jnp.float32
    )
    logits_ref[...] = lt.T

    erow = jax.lax.broadcasted_iota(jnp.int32, (_NUM_EXPERTS, _BR), 0)
    bits = jax.lax.bitcast_convert_type(lt, jnp.int32)
    mono = bits ^ ((bits >> 31) & jnp.int32(0x7FFFFFFF))
    key = (mono & jnp.int32(~63)) | (jnp.int32(63) - erow)

    vals, idxs = [], []
    for _ in range(_TOP_K):
        kmax = jnp.max(key, axis=0, keepdims=True)
        idxs.append(jnp.int32(63) - (kmax & jnp.int32(63)))
        kv = kmax & jnp.int32(~63)
        vbits = kv ^ ((kv >> 31) & jnp.int32(0x7FFFFFFF))
        vals.append(jax.lax.bitcast_convert_type(vbits, jnp.float32))
        key = jnp.where(key == kmax, jnp.int32(-(2**31)), key)

    topv = jnp.concatenate(vals, axis=0)
    topi = jnp.concatenate(idxs, axis=0)
    e = jnp.exp(topv - topv[:1])
    wts_ref[...] = (e / jnp.sum(e, axis=0, keepdims=True)).T
    idx_ref[...] = topi.T


def kernel(x, W):
    grid = (_N_TOKENS // _BR,)
    logits, wts, idx = pl.pallas_call(
        _gating_kernel,
        grid=grid,
        in_specs=[
            pl.BlockSpec((_BR, _HIDDEN), lambda i: (i, 0)),
            pl.BlockSpec((_NUM_EXPERTS, _HIDDEN), lambda i: (0, 0)),
        ],
        out_specs=[
            pl.BlockSpec((_BR, _NUM_EXPERTS), lambda i: (i, 0)),
            pl.BlockSpec((_BR, _TOP_K), lambda i: (i, 0)),
            pl.BlockSpec((_BR, _TOP_K), lambda i: (i, 0)),
        ],
        out_shape=[
            jax.ShapeDtypeStruct((_N_TOKENS, _NUM_EXPERTS), jnp.float32),
            jax.ShapeDtypeStruct((_N_TOKENS, _TOP_K), jnp.float32),
            jax.ShapeDtypeStruct((_N_TOKENS, _TOP_K), jnp.int32),
        ],
        compiler_params=pltpu.CompilerParams(
            dimension_semantics=("parallel",),
        ),
    )(x, W)
    return (wts, idx, logits)
